# baseline (speedup 1.0000x reference)
import functools

import jax
import jax.numpy as jnp
from jax import lax
from jax.experimental import pallas as pl
from jax.experimental.pallas import tpu as pltpu
from jax.experimental.pallas import tpu_sc as plsc

_VOCAB = 1000000
_PRETRAIN_DIM = 64
_EMBED_DIM = 32
_OOV_IDX = 999997
_B = 16384
_L = 50

_G = 4
_FUSE_BLK = 2000
_TAIL = _VOCAB - 8000


def _fuse_body(pt_ref, idt_ref, w4_ref, out_ref):
    i = pl.program_id(0)
    y = jax.lax.dot_general(
        pt_ref[...], w4_ref[...],
        dimension_numbers=(((1,), (0,)), ((), ())),
        preferred_element_type=jnp.float32,
    )
    row = i * _FUSE_BLK + jax.lax.broadcasted_iota(jnp.int32, (_FUSE_BLK, 128), 0)
    sub = jax.lax.broadcasted_iota(jnp.int32, (_FUSE_BLK, 128), 1) // _EMBED_DIM
    vocab_idx = row * _G + sub
    out_ref[...] = jnp.where(vocab_idx <= _OOV_IDX, y, -idt_ref[...])


def _fuse_tables(pretrain_g, id_tail_g, w4):
    n_rows = _VOCAB // _G
    grid = n_rows // _FUSE_BLK
    return pl.pallas_call(
        _fuse_body,
        grid=(grid,),
        in_specs=[
            pl.BlockSpec((_FUSE_BLK, _G * _PRETRAIN_DIM), lambda i: (i, 0)),
            pl.BlockSpec((_FUSE_BLK, _G * _EMBED_DIM), lambda i: (0, 0)),
            pl.BlockSpec((_G * _PRETRAIN_DIM, _G * _EMBED_DIM), lambda i: (0, 0)),
        ],
        out_specs=pl.BlockSpec((_FUSE_BLK, _G * _EMBED_DIM), lambda i: (i, 0)),
        out_shape=jax.ShapeDtypeStruct((n_rows, _G * _EMBED_DIM), jnp.float32),
    )(pretrain_g, id_tail_g, w4)


_NC, _NS = 2, 16
_NW = _NC * _NS
_RG = 8
_ROWS_PER_W = _B // _NW


def _make_gather():
    n_groups = _ROWS_PER_W // _RG
    mesh = plsc.VectorSubcoreMesh(core_axis_name="c", subcore_axis_name="s")

    @functools.partial(
        pl.kernel,
        mesh=mesh,
        out_type=jax.ShapeDtypeStruct((_B, _L, _EMBED_DIM), jnp.float32),
        scratch_types=[
            pltpu.VMEM((2, _RG, _L), jnp.int32),
            pltpu.VMEM((2, _RG, _L, _EMBED_DIM), jnp.float32),
            pltpu.SemaphoreType.DMA,
            pltpu.SemaphoreType.DMA,
        ],
        compiler_params=pltpu.CompilerParams(use_tc_tiling_on_sc=False),
    )
    def gather_k(t1_hbm, t2_hbm, idx_hbm, out_hbm, idx_v, rows_v, g_sem, o_sem):
        wid = lax.axis_index("s") * _NC + lax.axis_index("c")
        base = wid * _ROWS_PER_W

        sts = [None, None]
        prev = None
        for g in range(n_groups):
            slot = g % 2
            b0 = base + g * _RG
            if sts[slot] is not None:
                sts[slot].wait()
                sts[slot] = None
            pltpu.sync_copy(idx_hbm.at[pl.ds(b0, _RG)], idx_v.at[slot])
            cps1 = [
                pltpu.async_copy(
                    t1_hbm.at[idx_v.at[slot].at[j]],
                    rows_v.at[slot].at[j], g_sem)
                for j in range(_RG)
            ]
            if prev is not None:
                pg, ps, pcps2 = prev
                for cp in pcps2:
                    cp.wait()
                sts[ps] = pltpu.async_copy(
                    rows_v.at[ps], out_hbm.at[pl.ds(base + pg * _RG, _RG)],
                    o_sem)
            for cp in cps1:
                cp.wait()
            cps2 = [
                pltpu.async_copy(
                    t2_hbm.at[idx_v.at[slot].at[j]],
                    rows_v.at[slot].at[j], g_sem, add=True)
                for j in range(_RG)
            ]
            prev = (g, slot, cps2)
        pg, ps, pcps2 = prev
        for cp in pcps2:
            cp.wait()
        if sts[1 - ps] is not None:
            sts[1 - ps].wait()
        pltpu.async_copy(
            rows_v.at[ps], out_hbm.at[pl.ds(base + pg * _RG, _RG)],
            o_sem).wait()

    return gather_k


def kernel(inputs, pretrain_table, id_table, W_proj):
    w4 = jnp.kron(jnp.eye(_G, dtype=jnp.float32), W_proj.T)
    pretrain_g = pretrain_table.reshape(_VOCAB // _G, _G * _PRETRAIN_DIM)
    id_tail_g = id_table[_TAIL:].reshape(_FUSE_BLK, _G * _EMBED_DIM)

    proj = _fuse_tables(pretrain_g, id_tail_g, w4).reshape(_VOCAB, _EMBED_DIM)

    return _make_gather()(proj, id_table, inputs.astype(jnp.int32))


# speedup vs baseline: 1.0031x; 1.0031x over previous
import functools

import jax
import jax.numpy as jnp
from jax import lax
from jax.experimental import pallas as pl
from jax.experimental.pallas import tpu as pltpu
from jax.experimental.pallas import tpu_sc as plsc

_VOCAB = 1000000
_PRETRAIN_DIM = 64
_EMBED_DIM = 32
_OOV_IDX = 999997
_B = 16384
_L = 50

_FUSE_BLK = 8000
_TAIL = _VOCAB - _FUSE_BLK


def _fuse_body(pt_ref, idt_ref, w_ref, out_ref):
    i = pl.program_id(0)
    n = pl.num_programs(0)
    y = jax.lax.dot_general(
        pt_ref[...], w_ref[...],
        dimension_numbers=(((1,), (0,)), ((), ())),
        preferred_element_type=jnp.float32,
    )

    @pl.when(i != n - 1)
    def _():
        out_ref[...] = y

    @pl.when(i == n - 1)
    def _():
        row = i * _FUSE_BLK + jax.lax.broadcasted_iota(
            jnp.int32, (_FUSE_BLK, 128), 0)
        idt = idt_ref[...]
        idt128 = jnp.concatenate([idt, idt, idt, idt], axis=1)
        out_ref[...] = jnp.where(row <= _OOV_IDX, y, -idt128)


def _fuse_tables(pretrain_table, id_tail, w4r):
    grid = _VOCAB // _FUSE_BLK
    return pl.pallas_call(
        _fuse_body,
        grid=(grid,),
        in_specs=[
            pl.BlockSpec((_FUSE_BLK, _PRETRAIN_DIM), lambda i: (i, 0)),
            pl.BlockSpec((_FUSE_BLK, _EMBED_DIM), lambda i: (0, 0)),
            pl.BlockSpec((_PRETRAIN_DIM, 128), lambda i: (0, 0)),
        ],
        out_specs=pl.BlockSpec((_FUSE_BLK, 128), lambda i: (i, 0)),
        out_shape=jax.ShapeDtypeStruct((_VOCAB, 128), jnp.float32),
    )(pretrain_table, id_tail, w4r)


_NC, _NS = 2, 16
_NW = _NC * _NS
_RG = 8
_ROWS_PER_W = _B // _NW


def _make_gather():
    n_groups = _ROWS_PER_W // _RG
    mesh = plsc.VectorSubcoreMesh(core_axis_name="c", subcore_axis_name="s")

    @functools.partial(
        pl.kernel,
        mesh=mesh,
        out_type=jax.ShapeDtypeStruct((_B, _L, _EMBED_DIM), jnp.float32),
        scratch_types=[
            pltpu.VMEM((2, _RG, _L), jnp.int32),
            pltpu.VMEM((2, _RG, _L), jnp.int32),
            pltpu.VMEM((2, _RG, _L, _EMBED_DIM), jnp.float32),
            pltpu.SemaphoreType.DMA,
            pltpu.SemaphoreType.DMA,
        ],
        compiler_params=pltpu.CompilerParams(use_tc_tiling_on_sc=False),
    )
    def gather_k(t1_hbm, t2_hbm, idx_hbm, out_hbm,
                 idx4_v, idx_v, rows_v, g_sem, o_sem):
        wid = lax.axis_index("s") * _NC + lax.axis_index("c")
        base = wid * _ROWS_PER_W

        sts = [None, None]
        prev = None
        for g in range(n_groups):
            slot = g % 2
            b0 = base + g * _RG
            if sts[slot] is not None:
                sts[slot].wait()
                sts[slot] = None
            pltpu.sync_copy(idx_hbm.at[0].at[pl.ds(b0, _RG)],
                            idx4_v.at[slot])
            pltpu.sync_copy(idx_hbm.at[1].at[pl.ds(b0, _RG)],
                            idx_v.at[slot])
            cps1 = [
                pltpu.async_copy(
                    t1_hbm.at[idx4_v.at[slot].at[j]],
                    rows_v.at[slot].at[j], g_sem)
                for j in range(_RG)
            ]
            if prev is not None:
                pg, ps, pcps2 = prev
                for cp in pcps2:
                    cp.wait()
                sts[ps] = pltpu.async_copy(
                    rows_v.at[ps], out_hbm.at[pl.ds(base + pg * _RG, _RG)],
                    o_sem)
            for cp in cps1:
                cp.wait()
            cps2 = [
                pltpu.async_copy(
                    t2_hbm.at[idx_v.at[slot].at[j]],
                    rows_v.at[slot].at[j], g_sem, add=True)
                for j in range(_RG)
            ]
            prev = (g, slot, cps2)
        pg, ps, pcps2 = prev
        for cp in pcps2:
            cp.wait()
        if sts[1 - ps] is not None:
            sts[1 - ps].wait()
        pltpu.async_copy(
            rows_v.at[ps], out_hbm.at[pl.ds(base + pg * _RG, _RG)],
            o_sem).wait()

    return gather_k


def kernel(inputs, pretrain_table, id_table, W_proj):
    wt = W_proj.T
    w4r = jnp.concatenate([wt, wt, wt, wt], axis=1)
    id_tail = id_table[_TAIL:]

    proj = _fuse_tables(pretrain_table, id_tail, w4r)
    proj_lin = proj.reshape(4 * _VOCAB, _EMBED_DIM)

    idx = inputs.astype(jnp.int32)
    idx2 = jnp.stack([idx * 4, idx])

    return _make_gather()(proj_lin, id_table, idx2)
